# Initial kernel scaffold; baseline (speedup 1.0000x reference)
import jax
import jax.numpy as jnp
from jax.experimental import pallas as pl


def kernel(positions, types, node_attrs, W_embed, b_embed, W_msg1, W_upd1, W_msg2, W_upd2, W_out1, W_out2):
    raise NotImplementedError("write your pallas kernel here")


# speedup vs baseline: 8.0541x; 8.0541x over previous
import functools

import jax
import jax.numpy as jnp
from jax.experimental import pallas as pl
from jax.experimental.pallas import tpu as pltpu

_CUTOFF = 0.09
_NRBF = 16
_D = 128
_T = 128
_KMAX = 22


def _pair_body(a_ref, k_ref, pos_ref, post_ref, ht_ref, wr_ref,
               out_ref, *, nt):
    t = pl.program_id(0)
    j = pl.program_id(1)

    @pl.when(j == 0)
    def _init():
        out_ref[...] = jnp.zeros_like(out_ref)

    @pl.when(j < k_ref[t])
    def _compute():
        xd = pos_ref[:, 0:1]
        yd = pos_ref[:, 1:2]
        zd = pos_ref[:, 2:3]
        xs = post_ref[0:1, :]
        ys = post_ref[1:2, :]
        zs = post_ref[2:3, :]
        s = jnp.minimum(a_ref[t] + j, nt - 1)
        x2d = xd * xd + yd * yd + zd * zd
        x2s = xs * xs + ys * ys + zs * zs
        dot = jax.lax.dot_general(
            pos_ref[:, 0:3], post_ref[0:3, :], (((1,), (0,)), ((), ())),
            precision=jax.lax.Precision.HIGHEST,
            preferred_element_type=jnp.float32)
        d2m = x2d + x2s - 2.0 * dot
        gi = t * _T + jax.lax.broadcasted_iota(jnp.int32, (_T, _T), 0)
        gj = s * _T + jax.lax.broadcasted_iota(jnp.int32, (_T, _T), 1)
        penf = jnp.where((d2m < _CUTOFF * _CUTOFF) & (gi != gj), 0.0, -1e30)
        vx = xs - xd
        vy = ys - yd
        vz = zs - zd
        d2v = vx * vx + vy * vy + vz * vz
        d = jnp.sqrt(d2v + 1e-12)
        cen = (jax.lax.broadcasted_iota(jnp.int32, (_NRBF, 1, 1), 0)
               .astype(jnp.float32) * (_CUTOFF / (_NRBF - 1)))
        rbf = jnp.exp(-(((d[None, :, :] - cen) / (_CUTOFF / _NRBF)) ** 2))
        pen = penf[None, :, :]
        h_chan = jnp.broadcast_to(ht_ref[...][:, None, :], (_D, _T, _T))
        chan = jnp.concatenate(
            [h_chan, rbf, pen, jnp.zeros((7, _T, _T), jnp.float32)], axis=0)
        m = jax.nn.relu(jax.lax.dot_general(
            chan, wr_ref[...], (((0,), (0,)), ((), ())),
            preferred_element_type=jnp.float32))
        out_ref[...] = out_ref[...] + jnp.sum(m, axis=1)


def _embed_body(x_ref, w_ref, b_ref, o_ref):
    o_ref[...] = jax.nn.relu(
        jax.lax.dot_general(x_ref[...], w_ref[...], (((1,), (0,)), ((), ())),
                            preferred_element_type=jnp.float32)
        + b_ref[0:1, :])


def _mm_body(x_ref, w_ref, o_ref):
    o_ref[...] = jax.lax.dot_general(
        x_ref[...], w_ref[...], (((1,), (0,)), ((), ())),
        preferred_element_type=jnp.float32)


def _upd_body(h_ref, g_ref, w_ref, o_ref):
    hg = jnp.concatenate([h_ref[...], g_ref[...]], axis=1)
    o_ref[...] = jax.nn.relu(
        jax.lax.dot_general(hg, w_ref[...], (((1,), (0,)), ((), ())),
                            preferred_element_type=jnp.float32))


def _readout_body(h_ref, w1_ref, w2t_ref, o_ref, *, n, rt):
    i = pl.program_id(0)

    @pl.when(i == 0)
    def _init():
        o_ref[...] = jnp.zeros_like(o_ref)

    hid = jax.nn.relu(
        jax.lax.dot_general(h_ref[...], w1_ref[...], (((1,), (0,)), ((), ())),
                            preferred_element_type=jnp.float32))
    prod = hid * w2t_ref[0:1, :]
    rows = i * rt + jax.lax.broadcasted_iota(jnp.int32, prod.shape, 0)
    masked = jnp.where(rows < n, prod, 0.0)
    o_ref[...] = o_ref[...] + jnp.sum(masked)


def _row_tiles(npad):
    return 128


def _embed(x, w, b):
    npad, kdim = x.shape
    rt = _row_tiles(npad)
    return pl.pallas_call(
        _embed_body,
        grid=(npad // rt,),
        in_specs=[pl.BlockSpec((rt, kdim), lambda i: (i, 0)),
                  pl.BlockSpec(w.shape, lambda i: (0, 0)),
                  pl.BlockSpec(b.shape, lambda i: (0, 0))],
        out_specs=pl.BlockSpec((rt, w.shape[1]), lambda i: (i, 0)),
        out_shape=jax.ShapeDtypeStruct((npad, w.shape[1]), jnp.float32),
    )(x, w, b)


def _mm(x, w):
    npad = x.shape[0]
    rt = _row_tiles(npad)
    return pl.pallas_call(
        _mm_body,
        grid=(npad // rt,),
        in_specs=[pl.BlockSpec((rt, x.shape[1]), lambda i: (i, 0)),
                  pl.BlockSpec(w.shape, lambda i: (0, 0))],
        out_specs=pl.BlockSpec((rt, w.shape[1]), lambda i: (i, 0)),
        out_shape=jax.ShapeDtypeStruct((npad, w.shape[1]), jnp.float32),
    )(x, w)


def _update(h, g, w):
    npad = h.shape[0]
    rt = _row_tiles(npad)
    return pl.pallas_call(
        _upd_body,
        grid=(npad // rt,),
        in_specs=[pl.BlockSpec((rt, _D), lambda i: (i, 0)),
                  pl.BlockSpec((rt, _D), lambda i: (i, 0)),
                  pl.BlockSpec(w.shape, lambda i: (0, 0))],
        out_specs=pl.BlockSpec((rt, _D), lambda i: (i, 0)),
        out_shape=jax.ShapeDtypeStruct((npad, _D), jnp.float32),
    )(h, g, w)


def _readout(h, w1, w2t, n):
    npad = h.shape[0]
    rt = _row_tiles(npad)
    out = pl.pallas_call(
        functools.partial(_readout_body, n=n, rt=rt),
        grid=(npad // rt,),
        in_specs=[pl.BlockSpec((rt, _D), lambda i: (i, 0)),
                  pl.BlockSpec(w1.shape, lambda i: (0, 0)),
                  pl.BlockSpec(w2t.shape, lambda i: (0, 0))],
        out_specs=pl.BlockSpec((8, 128), lambda i: (0, 0)),
        out_shape=jax.ShapeDtypeStruct((8, 128), jnp.float32),
    )(h, w1, w2t)
    return out[0, 0].reshape(1)


def _pair_agg(a, k, pos8, post, ht, wr, nt, kmax):
    grid_spec = pltpu.PrefetchScalarGridSpec(
        num_scalar_prefetch=2,
        grid=(nt, kmax),
        in_specs=[
            pl.BlockSpec((_T, 8), lambda t, j, a, k: (t, 0)),
            pl.BlockSpec((8, _T),
                         lambda t, j, a, k: (0, jnp.minimum(a[t] + j, nt - 1))),
            pl.BlockSpec((_D, _T),
                         lambda t, j, a, k: (0, jnp.minimum(a[t] + j, nt - 1))),
            pl.BlockSpec(wr.shape, lambda t, j, a, k: (0, 0)),
        ],
        out_specs=pl.BlockSpec((_T, _D), lambda t, j, a, k: (t, 0)),
    )
    return pl.pallas_call(
        functools.partial(_pair_body, nt=nt),
        grid_spec=grid_spec,
        out_shape=jax.ShapeDtypeStruct((nt * _T, _D), jnp.float32),
    )(a, k, pos8, post, ht, wr)


def kernel(positions, types, node_attrs, W_embed, b_embed, W_msg1, W_upd1,
           W_msg2, W_upd2, W_out1, W_out2):
    del types
    positions = positions.astype(jnp.float32)
    n = positions.shape[0]
    npad = -(-n // _T) * _T
    nt = npad // _T
    kmax = min(_KMAX, nt)

    attrs0 = node_attrs.astype(jnp.float32)
    ka = attrs0.shape[1]
    ops = (positions[:, 0], positions[:, 1], positions[:, 2]) + tuple(
        attrs0[:, i] for i in range(ka))
    sops = jax.lax.sort(ops, dimension=0, num_keys=1)
    pos = jnp.concatenate(
        [jnp.stack(sops[0:3], axis=1),
         jnp.full((npad - n, 3), 2.0, jnp.float32)], axis=0)
    attrs = jnp.stack(sops[3:], axis=1)
    kpad = -(-ka // 8) * 8
    attrs = jnp.concatenate(
        [attrs, jnp.zeros((npad - n, ka), jnp.float32)], axis=0)
    attrs = jnp.concatenate(
        [attrs, jnp.zeros((npad, kpad - ka), jnp.float32)], axis=1)
    w_emb = jnp.concatenate(
        [W_embed.astype(jnp.float32),
         jnp.zeros((kpad - ka, _D), jnp.float32)], axis=0)
    pos8 = jnp.concatenate([pos, jnp.zeros((npad, 5), jnp.float32)], axis=1)
    post = jnp.concatenate([pos.T, jnp.zeros((5, npad), jnp.float32)], axis=0)

    xs = pos[:, 0]
    xt = xs.reshape(nt, _T)
    lo = jnp.searchsorted(xs, xt[:, 0] - _CUTOFF, side="left")
    hi = jnp.searchsorted(xs, xt[:, -1] + _CUTOFF, side="right")
    a = (lo // _T).astype(jnp.int32)
    k = ((hi - a * _T + (_T - 1)) // _T).astype(jnp.int32)
    k = jnp.minimum(k, kmax).astype(jnp.int32)

    bvec = jnp.concatenate(
        [b_embed.astype(jnp.float32).reshape(1, _D),
         jnp.zeros((7, _D), jnp.float32)], axis=0)

    h = _embed(attrs, w_emb, bvec)
    for wm, wu in ((W_msg1, W_upd1), (W_msg2, W_upd2)):
        wm = wm.astype(jnp.float32)
        wu = wu.astype(jnp.float32)
        ht = h.T
        wr_aug = jnp.concatenate(
            [wm, jnp.ones((1, _D), jnp.float32),
             jnp.zeros((7, _D), jnp.float32)], axis=0)
        agg = _pair_agg(a, k, pos8, post, ht, wr_aug, nt, kmax)
        h = _update(h, agg, wu)

    w2t = jnp.concatenate(
        [W_out2.astype(jnp.float32).T, jnp.zeros((7, 64), jnp.float32)], axis=0)
    return _readout(h, W_out1.astype(jnp.float32), w2t, n)
